# baseline (speedup 1.0000x reference)
import functools

import jax
import jax.numpy as jnp
from jax import lax
from jax.experimental import pallas as pl
from jax.experimental.pallas import tpu as pltpu
from jax.experimental.pallas import tpu_sc as plsc

N_NODES = 10000
N_EDGES = 320000
D_FEAT = 128
HF = D_FEAT // 2

NC = 2
NS = 16
W = 80
EPW = N_EDGES // NS
WPW = EPW // W
NBUF = 10
CHUNK = 50
NCHUNK = WPW // CHUNK

ROWS_MAIN = 624
TAIL_BASE = NS * ROWS_MAIN
TAIL_ROWS = N_NODES - TAIL_BASE


def _sc_segment_halves(img_halves, src1d, dst4d):
    mesh = plsc.VectorSubcoreMesh(core_axis_name="c", subcore_axis_name="s")

    @functools.partial(
        pl.kernel,
        out_type=jax.ShapeDtypeStruct((NC, N_NODES, HF), jnp.float32),
        mesh=mesh,
        compiler_params=pltpu.CompilerParams(use_tc_tiling_on_sc=False),
        scratch_types=[
            pltpu.VMEM_SHARED((N_NODES, HF), jnp.float32),
            pltpu.VMEM((CHUNK * W,), jnp.int32),
            pltpu.VMEM((CHUNK * W,), jnp.int32),
            pltpu.VMEM((CHUNK, W), jnp.int32),
            pltpu.VMEM((CHUNK, W), jnp.int32),
            pltpu.VMEM((NBUF, W, HF), jnp.float32),
            pltpu.SemaphoreType.DMA,
            pltpu.SemaphoreType.DMA,
            pltpu.SemaphoreType.DMA,
            pltpu.SemaphoreType.DMA,
            pltpu.SemaphoreType.DMA,
            pltpu.SemaphoreType.DMA,
            pltpu.SemaphoreType.DMA,
            pltpu.SemaphoreType.DMA,
            pltpu.SemaphoreType.DMA,
            pltpu.SemaphoreType.DMA,
            pltpu.SemaphoreType.DMA,
        ],
    )
    def k(img_hbm, src_hbm, dst_hbm, out_hbm, acc, src0, src1, dst0, dst1,
          ring, g0, g1, g2, g3, g4, g5, g6, g7, g8, g9, isem):
        c = lax.axis_index("c")
        s = lax.axis_index("s")

        gsem = [g0, g1, g2, g3, g4, g5, g6, g7, g8, g9]
        srcb = [src0, src1]
        dstb = [dst0, dst1]
        table = img_hbm.at[c]

        zero = jnp.zeros((16,), jnp.float32)

        @pl.loop(0, W)
        def _(i):
            @pl.loop(0, HF, step=16)
            def _(j):
                ring[0, i, pl.ds(j, 16)] = zero

        row_base = s * ROWS_MAIN

        @pl.loop(0, ROWS_MAIN - 64, step=W)
        def _(r):
            pltpu.sync_copy(ring.at[0], acc.at[pl.ds(row_base + r, W)])

        pltpu.sync_copy(ring.at[0].at[pl.ds(0, 64)],
                        acc.at[pl.ds(row_base + 560, 64)])

        @pl.when(s == NS - 1)
        def _():
            pltpu.sync_copy(ring.at[0].at[pl.ds(0, TAIL_ROWS)],
                            acc.at[pl.ds(TAIL_BASE, TAIL_ROWS)])

        def load_chunk(ci, sb, db, sem):
            pltpu.async_copy(
                src_hbm.at[pl.ds(s * EPW + ci * (CHUNK * W), CHUNK * W)],
                sb, sem)
            pltpu.async_copy(dst_hbm.at[s, ci], db, sem)

        def wait_chunk(ci, sb, db, sem):
            pltpu.make_async_copy(
                src_hbm.at[pl.ds(s * EPW + ci * (CHUNK * W), CHUNK * W)],
                sb, sem).wait()
            pltpu.make_async_copy(dst_hbm.at[s, ci], db, sem).wait()

        load_chunk(0, src0, dst0, isem)
        wait_chunk(0, src0, dst0, isem)

        plsc.subcore_barrier()

        def start_gather(sb, widx, b):
            pltpu.async_copy(table.at[sb.at[pl.ds(widx * W, W)]],
                             ring.at[b], gsem[b])

        def wait_gather(b):
            pltpu.make_async_copy(table.at[src0.at[pl.ds(0, W)]],
                                  ring.at[b], gsem[b]).wait()

        def scatter_add(db, widx, b):
            pltpu.sync_copy(ring.at[b], acc.at[db.at[widx]], add=True)

        for b in range(NBUF):
            start_gather(src0, b, b)

        for ci in range(NCHUNK):
            sb, db = srcb[ci % 2], dstb[ci % 2]
            sn, dn = srcb[(ci + 1) % 2], dstb[(ci + 1) % 2]
            if ci + 1 < NCHUNK:
                load_chunk(ci + 1, sn, dn, isem)

            @pl.loop(0, CHUNK - NBUF, step=NBUF)
            def _(i):
                for b in range(NBUF):
                    wait_gather(b)
                    scatter_add(db, i + b, b)
                    start_gather(sb, i + b + NBUF, b)

            if ci + 1 < NCHUNK:
                wait_chunk(ci + 1, sn, dn, isem)
                for b in range(NBUF):
                    wait_gather(b)
                    scatter_add(db, CHUNK - NBUF + b, b)
                    start_gather(sn, b, b)
            else:
                for b in range(NBUF):
                    wait_gather(b)
                    scatter_add(db, CHUNK - NBUF + b, b)

        plsc.subcore_barrier()

        pltpu.sync_copy(
            acc.at[pl.ds(row_base, ROWS_MAIN)],
            out_hbm.at[c].at[pl.ds(row_base, ROWS_MAIN)],
        )

        @pl.when(s == NS - 1)
        def _():
            pltpu.sync_copy(
                acc.at[pl.ds(TAIL_BASE, TAIL_ROWS)],
                out_hbm.at[c].at[pl.ds(TAIL_BASE, TAIL_ROWS)],
            )

    return k(img_halves, src1d, dst4d)


def _tc_concat(halves):
    def body(p_ref, o_ref):
        o_ref[...] = jnp.concatenate([p_ref[0], p_ref[1]], axis=-1)

    blk = 2000
    return pl.pallas_call(
        body,
        out_shape=jax.ShapeDtypeStruct((N_NODES, D_FEAT), jnp.float32),
        grid=(N_NODES // blk,),
        in_specs=[pl.BlockSpec((NC, blk, HF), lambda i: (0, i, 0))],
        out_specs=pl.BlockSpec((blk, D_FEAT), lambda i: (i, 0)),
    )(halves)


@jax.jit
def kernel(image, edge_index):
    img_halves = jnp.stack([image[:, :HF], image[:, HF:]])
    src1d = edge_index[0]
    dst4d = edge_index[1].reshape(NS, NCHUNK, CHUNK, W)
    halves = _sc_segment_halves(img_halves, src1d, dst4d)
    mailbox_agg = _tc_concat(halves)
    return (image, mailbox_agg)


# speedup vs baseline: 1.1672x; 1.1672x over previous
import functools

import jax
import jax.numpy as jnp
from jax import lax
from jax.experimental import pallas as pl
from jax.experimental.pallas import tpu as pltpu
from jax.experimental.pallas import tpu_sc as plsc

N_NODES = 10000
N_EDGES = 320000
D_FEAT = 128

NC = 2
NS = 16
NW = NC * NS
W = 40
EPW = N_EDGES // NW
WPW = EPW // W
NBUF = 5
CHUNK = 50
NCHUNK = WPW // CHUNK

ROWS_MAIN = 624
TAIL_BASE = NS * ROWS_MAIN
TAIL_ROWS = N_NODES - TAIL_BASE


def _sc_segment_partials(image, src1d, dst4d):
    mesh = plsc.VectorSubcoreMesh(core_axis_name="c", subcore_axis_name="s")

    @functools.partial(
        pl.kernel,
        out_type=jax.ShapeDtypeStruct((NC, N_NODES, D_FEAT), jnp.float32),
        mesh=mesh,
        compiler_params=pltpu.CompilerParams(use_tc_tiling_on_sc=False),
        scratch_types=[
            pltpu.VMEM_SHARED((N_NODES, D_FEAT), jnp.float32),
            pltpu.VMEM((CHUNK * W,), jnp.int32),
            pltpu.VMEM((CHUNK * W,), jnp.int32),
            pltpu.VMEM((CHUNK, W), jnp.int32),
            pltpu.VMEM((CHUNK, W), jnp.int32),
            pltpu.VMEM((NBUF, W, D_FEAT), jnp.float32),
            pltpu.SemaphoreType.DMA,
            pltpu.SemaphoreType.DMA,
            pltpu.SemaphoreType.DMA,
            pltpu.SemaphoreType.DMA,
            pltpu.SemaphoreType.DMA,
            pltpu.SemaphoreType.DMA,
        ],
    )
    def k(image_hbm, src_hbm, dst_hbm, out_hbm, acc, src0, src1, dst0, dst1,
          ring, g0, g1, g2, g3, g4, isem):
        c = lax.axis_index("c")
        s = lax.axis_index("s")
        wid = c * NS + s

        gsem = [g0, g1, g2, g3, g4]
        srcb = [src0, src1]
        dstb = [dst0, dst1]

        zero = jnp.zeros((16,), jnp.float32)

        @pl.loop(0, W)
        def _(i):
            @pl.loop(0, D_FEAT, step=16)
            def _(j):
                ring[0, i, pl.ds(j, 16)] = zero

        row_base = s * ROWS_MAIN

        @pl.loop(0, ROWS_MAIN - 24, step=W)
        def _(r):
            pltpu.sync_copy(ring.at[0], acc.at[pl.ds(row_base + r, W)])

        pltpu.sync_copy(ring.at[0].at[pl.ds(0, 24)],
                        acc.at[pl.ds(row_base + 600, 24)])

        @pl.when(s == NS - 1)
        def _():
            pltpu.sync_copy(ring.at[0].at[pl.ds(0, TAIL_ROWS)],
                            acc.at[pl.ds(TAIL_BASE, TAIL_ROWS)])

        def load_chunk(ci, sb, db, sem):
            pltpu.async_copy(
                src_hbm.at[pl.ds(wid * EPW + ci * (CHUNK * W), CHUNK * W)],
                sb, sem)
            pltpu.async_copy(dst_hbm.at[wid, ci], db, sem)

        def wait_chunk(ci, sb, db, sem):
            pltpu.make_async_copy(
                src_hbm.at[pl.ds(wid * EPW + ci * (CHUNK * W), CHUNK * W)],
                sb, sem).wait()
            pltpu.make_async_copy(dst_hbm.at[wid, ci], db, sem).wait()

        load_chunk(0, src0, dst0, isem)
        wait_chunk(0, src0, dst0, isem)

        plsc.subcore_barrier()

        def start_gather(sb, widx, b):
            pltpu.async_copy(image_hbm.at[sb.at[pl.ds(widx * W, W)]],
                             ring.at[b], gsem[b])

        def wait_gather(b):
            pltpu.make_async_copy(image_hbm.at[src0.at[pl.ds(0, W)]],
                                  ring.at[b], gsem[b]).wait()

        def scatter_add(db, widx, b):
            pltpu.sync_copy(ring.at[b], acc.at[db.at[widx]], add=True)

        for b in range(NBUF):
            start_gather(src0, b, b)

        for ci in range(NCHUNK):
            sb, db = srcb[ci % 2], dstb[ci % 2]
            sn, dn = srcb[(ci + 1) % 2], dstb[(ci + 1) % 2]
            if ci + 1 < NCHUNK:
                load_chunk(ci + 1, sn, dn, isem)

            @pl.loop(0, CHUNK - NBUF, step=NBUF)
            def _(i):
                for b in range(NBUF):
                    wait_gather(b)
                    scatter_add(db, i + b, b)
                    start_gather(sb, i + b + NBUF, b)

            if ci + 1 < NCHUNK:
                wait_chunk(ci + 1, sn, dn, isem)
                for b in range(NBUF):
                    wait_gather(b)
                    scatter_add(db, CHUNK - NBUF + b, b)
                    start_gather(sn, b, b)
            else:
                for b in range(NBUF):
                    wait_gather(b)
                    scatter_add(db, CHUNK - NBUF + b, b)

        plsc.subcore_barrier()

        pltpu.sync_copy(
            acc.at[pl.ds(row_base, ROWS_MAIN)],
            out_hbm.at[c].at[pl.ds(row_base, ROWS_MAIN)],
        )

        @pl.when(s == NS - 1)
        def _():
            pltpu.sync_copy(
                acc.at[pl.ds(TAIL_BASE, TAIL_ROWS)],
                out_hbm.at[c].at[pl.ds(TAIL_BASE, TAIL_ROWS)],
            )

    return k(image, src1d, dst4d)


def _tc_combine(partials):
    def body(p_ref, o_ref):
        o_ref[...] = p_ref[0] + p_ref[1]

    blk = 2000
    return pl.pallas_call(
        body,
        out_shape=jax.ShapeDtypeStruct((N_NODES, D_FEAT), jnp.float32),
        grid=(N_NODES // blk,),
        in_specs=[pl.BlockSpec((NC, blk, D_FEAT), lambda i: (0, i, 0))],
        out_specs=pl.BlockSpec((blk, D_FEAT), lambda i: (i, 0)),
    )(partials)


@jax.jit
def kernel(image, edge_index):
    src1d = edge_index[0]
    dst4d = edge_index[1].reshape(NW, NCHUNK, CHUNK, W)
    partials = _sc_segment_partials(image, src1d, dst4d)
    mailbox_agg = _tc_combine(partials)
    return (image, mailbox_agg)
